# baseline (speedup 1.0000x reference)
import functools

import jax
import jax.numpy as jnp
from jax import lax
from jax.experimental import pallas as pl
from jax.experimental.pallas import tpu as pltpu
from jax.experimental.pallas import tpu_sc as plsc

B = 16384
V = 1000000
F = 64
NC = 2
NS = 16
NW = NC * NS
BPW = B // NW
CHUNK = 128
NCHUNK = BPW // CHUNK
BLK = 16
NBLK = BPW // BLK


def _sc_body(user_hbm, item_hbm, ctx_hbm, table_hbm, out_hbm,
             idx_v, rows_u, rows_i, rows_c, outbuf, sem):
    wid = lax.axis_index("s") * NC + lax.axis_index("c")
    base = wid * BPW

    copies = []
    for j in range(NCHUNK):
        src = pl.ds(base + j * CHUNK, CHUNK)
        copies.append(pltpu.async_copy(user_hbm.at[src], idx_v.at[0, j], sem))
        copies.append(pltpu.async_copy(item_hbm.at[src], idx_v.at[1, j], sem))
        copies.append(pltpu.async_copy(ctx_hbm.at[src], idx_v.at[2, j], sem))
    for c in copies:
        c.wait()

    gathers = []
    for j in range(NCHUNK):
        dst = pl.ds(j * CHUNK, CHUNK)
        gathers.append(pltpu.async_copy(
            table_hbm.at[idx_v.at[0, j]], rows_u.at[dst, :], sem))
        gathers.append(pltpu.async_copy(
            table_hbm.at[idx_v.at[1, j]], rows_i.at[dst, :], sem))
        gathers.append(pltpu.async_copy(
            table_hbm.at[idx_v.at[2, j]], rows_c.at[dst, :], sem))
    for g in gathers:
        g.wait()

    zero = jnp.zeros((16,), jnp.float32)

    def blk_body(blk, carry):
        row_ids = blk * BLK + lax.iota(jnp.int32, 16)
        accs = [zero, zero, zero, zero]
        for f in range(F):
            col = jnp.full((16,), f, jnp.int32)
            u = plsc.load_gather(rows_u, [row_ids, col])
            it = plsc.load_gather(rows_i, [row_ids, col])
            ct = plsc.load_gather(rows_c, [row_ids, col])
            accs[f % 4] = accs[f % 4] + u * it * ct
        outbuf[pl.ds(blk * BLK, BLK)] = (accs[0] + accs[1]) + (accs[2] + accs[3])
        return carry

    lax.fori_loop(0, NBLK, blk_body, 0)

    pltpu.sync_copy(outbuf, out_hbm.at[pl.ds(base, BPW)])


@functools.partial(
    pl.kernel,
    out_type=jax.ShapeDtypeStruct((B,), jnp.float32),
    mesh=plsc.VectorSubcoreMesh(core_axis_name="c", subcore_axis_name="s"),
    compiler_params=pltpu.CompilerParams(
        needs_layout_passes=False, use_tc_tiling_on_sc=False),
    scratch_types=[
        pltpu.VMEM((3, NCHUNK, CHUNK), jnp.int32),
        pltpu.VMEM((BPW, F), jnp.float32),
        pltpu.VMEM((BPW, F), jnp.float32),
        pltpu.VMEM((BPW, F), jnp.float32),
        pltpu.VMEM((BPW,), jnp.float32),
        pltpu.SemaphoreType.DMA,
    ],
)
def _pointmf_sc(user_hbm, item_hbm, ctx_hbm, table_hbm, out_hbm,
                idx_v, rows_u, rows_i, rows_c, outbuf, sem):
    _sc_body(user_hbm, item_hbm, ctx_hbm, table_hbm, out_hbm,
             idx_v, rows_u, rows_i, rows_c, outbuf, sem)


def kernel(user, item, context, table):
    return _pointmf_sc(user.astype(jnp.int32), item.astype(jnp.int32),
                       context.astype(jnp.int32), table)


# speedup vs baseline: 1.7886x; 1.7886x over previous
import functools

import jax
import jax.numpy as jnp
from jax import lax
from jax.experimental import pallas as pl
from jax.experimental.pallas import tpu as pltpu
from jax.experimental.pallas import tpu_sc as plsc

B = 16384
V = 1000000
F = 64
NC = 2
NS = 16
NW = NC * NS
BPW = B // NW
C = 16
NCH = BPW // C


def _start_fetches(table_hbm, idxs, bufs, b, chunk, sem):
    for t in range(3):
        vidx = idxs[t][pl.ds(chunk * C, C)]
        for i in range(C):
            pltpu.async_copy(table_hbm.at[vidx[i]], bufs.at[b, t, i], sem)


def _drain_fetches(table_hbm, bufs, b, sem):
    for t in range(3):
        for i in range(C):
            pltpu.make_async_copy(
                table_hbm.at[0], bufs.at[b, t, i], sem).wait()


def _compute_chunk(bufs, b, chunk, outbuf):
    lane = lax.iota(jnp.int32, 16)
    tot = jnp.zeros((16,), jnp.float32)
    for i in range(C):
        rows = [[bufs[b, t, i, pl.ds(j * 16, 16)] for j in range(F // 16)]
                for t in range(3)]
        parts = [rows[0][j] * rows[1][j] * rows[2][j] for j in range(F // 16)]
        s = (parts[0] + parts[1]) + (parts[2] + parts[3])
        tot = jnp.where(lane == i, jnp.sum(s), tot)
    outbuf[pl.ds(chunk * C, C)] = tot


def _sc_body(user_hbm, item_hbm, ctx_hbm, table_hbm, out_hbm,
             idx_u, idx_i, idx_c, bufs, outbuf, sem_idx, sem0, sem1):
    idxs = (idx_u, idx_i, idx_c)
    wid = lax.axis_index("s") * NC + lax.axis_index("c")
    base = wid * BPW

    cps = [
        pltpu.async_copy(user_hbm.at[pl.ds(base, BPW)], idx_u, sem_idx),
        pltpu.async_copy(item_hbm.at[pl.ds(base, BPW)], idx_i, sem_idx),
        pltpu.async_copy(ctx_hbm.at[pl.ds(base, BPW)], idx_c, sem_idx),
    ]
    for cp in cps:
        cp.wait()

    _start_fetches(table_hbm, idxs, bufs, 0, 0, sem0)

    def pipe(k, carry):
        g = k * 2
        _start_fetches(table_hbm, idxs, bufs, 1, g + 1, sem1)
        _drain_fetches(table_hbm, bufs, 0, sem0)
        _compute_chunk(bufs, 0, g, outbuf)

        @pl.when(g + 2 < NCH)
        def _():
            _start_fetches(table_hbm, idxs, bufs, 0, g + 2, sem0)

        _drain_fetches(table_hbm, bufs, 1, sem1)
        _compute_chunk(bufs, 1, g + 1, outbuf)
        return carry

    lax.fori_loop(0, NCH // 2, pipe, 0)

    pltpu.sync_copy(outbuf, out_hbm.at[pl.ds(base, BPW)])


@functools.partial(
    pl.kernel,
    out_type=jax.ShapeDtypeStruct((B,), jnp.float32),
    mesh=plsc.VectorSubcoreMesh(core_axis_name="c", subcore_axis_name="s"),
    compiler_params=pltpu.CompilerParams(
        needs_layout_passes=False, use_tc_tiling_on_sc=True),
    scratch_types=[
        pltpu.VMEM((BPW,), jnp.int32),
        pltpu.VMEM((BPW,), jnp.int32),
        pltpu.VMEM((BPW,), jnp.int32),
        pltpu.VMEM((2, 3, C, F), jnp.float32),
        pltpu.VMEM((BPW,), jnp.float32),
        pltpu.SemaphoreType.DMA,
        pltpu.SemaphoreType.DMA,
        pltpu.SemaphoreType.DMA,
    ],
)
def _pointmf_sc(user_hbm, item_hbm, ctx_hbm, table_hbm, out_hbm,
                idx_u, idx_i, idx_c, bufs, outbuf, sem_idx, sem0, sem1):
    _sc_body(user_hbm, item_hbm, ctx_hbm, table_hbm, out_hbm,
             idx_u, idx_i, idx_c, bufs, outbuf, sem_idx, sem0, sem1)


def kernel(user, item, context, table):
    return _pointmf_sc(user.astype(jnp.int32), item.astype(jnp.int32),
                       context.astype(jnp.int32), table)
